# Initial kernel scaffold; baseline (speedup 1.0000x reference)
import functools

import jax
import jax.numpy as jnp
from jax import lax
from jax.experimental import pallas as pl
from jax.experimental.pallas import tpu as pltpu
from jax.experimental.pallas import tpu_sc as plsc

N = 10000
NP = 10240
E = 320000
D_IN = 128
H = 64
K = 8
T = 4
D_OUT = 4

C = 128
TILES = 16
CORES = 2
WORKERS = CORES * TILES
EP16 = E // TILES
NC16 = -(-EP16 // C)
EP32 = E // WORKERS
NC32 = -(-EP32 // C)
ZR = NP // TILES

_mesh = plsc.VectorSubcoreMesh(core_axis_name="c", subcore_axis_name="s")


@functools.partial(
    pl.kernel,
    out_type=jax.ShapeDtypeStruct((CORES, NP, 16), jnp.float32),
    mesh=_mesh,
    scratch_types=[
        pltpu.VMEM((NC32, C), jnp.int32),
        pltpu.VMEM((C, 16), jnp.float32),
        pltpu.VMEM_SHARED((NP, 16), jnp.float32),
    ],
)
def _sc_degree(dst32, ones_hbm, z16, deg_out, idx_v, ones_v, acc_sh):
    c = lax.axis_index("c")
    s = lax.axis_index("s")
    wid = c * TILES + s
    pltpu.sync_copy(dst32.at[wid], idx_v)
    pltpu.sync_copy(ones_hbm, ones_v)
    pltpu.sync_copy(z16, acc_sh.at[pl.ds(s * ZR, ZR)])
    plsc.subcore_barrier()

    def chunk(g, carry):
        pltpu.sync_copy(ones_v, acc_sh.at[idx_v.at[g]], add=True)
        return carry

    lax.fori_loop(0, NC32, chunk, 0)
    plsc.subcore_barrier()
    pltpu.sync_copy(acc_sh.at[pl.ds(s * ZR, ZR)], deg_out.at[c, pl.ds(s * ZR, ZR)])


@functools.partial(
    pl.kernel,
    out_type=jax.ShapeDtypeStruct((CORES, NP, H), jnp.float32),
    mesh=_mesh,
    scratch_types=[
        pltpu.VMEM((NC32, C), jnp.int32),
        pltpu.VMEM((NC32, C), jnp.int32),
        pltpu.VMEM((C, H), jnp.float32),
        pltpu.VMEM_SHARED((NP, H), jnp.float32),
        pltpu.SemaphoreType.DMA,
    ],
)
def _sc_prop0(cur_hbm, src32, dst32, z64, out, idx_s, idx_d, rows_v, acc_sh, sem):
    c = lax.axis_index("c")
    s = lax.axis_index("s")
    wid = c * TILES + s
    pltpu.sync_copy(src32.at[wid], idx_s)
    pltpu.sync_copy(dst32.at[wid], idx_d)
    pltpu.sync_copy(z64, acc_sh.at[pl.ds(s * ZR, ZR)])
    plsc.subcore_barrier()

    def chunk(g, carry):
        pltpu.async_copy(cur_hbm.at[idx_s.at[g]], rows_v, sem).wait()
        pltpu.sync_copy(rows_v, acc_sh.at[idx_d.at[g]], add=True)
        return carry

    lax.fori_loop(0, NC32, chunk, 0)
    plsc.subcore_barrier()
    pltpu.sync_copy(acc_sh.at[pl.ds(s * ZR, ZR)], out.at[c, pl.ds(s * ZR, ZR)])


@functools.partial(
    pl.kernel,
    out_type=jax.ShapeDtypeStruct((K, NP, H), jnp.float32),
    mesh=_mesh,
    scratch_types=[
        pltpu.VMEM((NC16, C), jnp.int32),
        pltpu.VMEM((NC16, C), jnp.int32),
        pltpu.VMEM((C, H), jnp.float32),
        pltpu.VMEM_SHARED((NP, H), jnp.float32),
        pltpu.SemaphoreType.DMA,
    ],
)
def _sc_prop(cur_hbm, src16, dst16, z64, out, idx_s, idx_d, rows_v, acc_sh, sem):
    c = lax.axis_index("c")
    s = lax.axis_index("s")
    pltpu.sync_copy(src16.at[s], idx_s)
    pltpu.sync_copy(dst16.at[s], idx_d)
    for kk in range(K // CORES):
        k = c * (K // CORES) + kk
        pltpu.sync_copy(z64, acc_sh.at[pl.ds(s * ZR, ZR)])
        plsc.subcore_barrier()

        def chunk(g, carry):
            pltpu.async_copy(cur_hbm.at[k].at[idx_s.at[g]], rows_v, sem).wait()
            pltpu.sync_copy(rows_v, acc_sh.at[idx_d.at[g]], add=True)
            return carry

        lax.fori_loop(0, NC16, chunk, 0)
        plsc.subcore_barrier()
        pltpu.sync_copy(acc_sh.at[pl.ds(s * ZR, ZR)], out.at[k, pl.ds(s * ZR, ZR)])


def _leaky(v):
    return jnp.where(v >= 0, v, 0.2 * v)


BN = 2048


def _pre_body(x_ref, w1, b1, w2, b2, deg_ref, x0_out, curs_out, dinv_out):
    xb = x_ref[...]
    xb = jnp.where(jnp.isnan(xb), 0.0, xb)
    h1 = _leaky(jnp.dot(xb, w1[...], preferred_element_type=jnp.float32) + b1[...])
    h2 = _leaky(jnp.dot(h1, w2[...], preferred_element_type=jnp.float32) + b2[...])
    deg = deg_ref[0, :, 0:1] + deg_ref[1, :, 0:1]
    dinv = jnp.where(deg > 0, lax.rsqrt(jnp.where(deg > 0, deg, 1.0)), 0.0)
    x0_out[...] = h2
    dinv_out[...] = jnp.broadcast_to(dinv, (BN, H))
    curs_out[...] = h2 * dinv


def _tc_pre(xp, W1, b1, W2, b2, deg2):
    nb = NP // BN
    return pl.pallas_call(
        _pre_body,
        grid=(nb,),
        in_specs=[
            pl.BlockSpec((BN, D_IN), lambda i: (i, 0)),
            pl.BlockSpec((D_IN, H), lambda i: (0, 0)),
            pl.BlockSpec((1, H), lambda i: (0, 0)),
            pl.BlockSpec((H, H), lambda i: (0, 0)),
            pl.BlockSpec((1, H), lambda i: (0, 0)),
            pl.BlockSpec((CORES, BN, 16), lambda i: (0, i, 0)),
        ],
        out_specs=[
            pl.BlockSpec((BN, H), lambda i: (i, 0)),
            pl.BlockSpec((BN, H), lambda i: (i, 0)),
            pl.BlockSpec((BN, H), lambda i: (i, 0)),
        ],
        out_shape=[
            jax.ShapeDtypeStruct((NP, H), jnp.float32),
            jax.ShapeDtypeStruct((NP, H), jnp.float32),
            jax.ShapeDtypeStruct((NP, H), jnp.float32),
        ],
    )(xp, W1, b1, W2, b2, deg2)


def _stage_body(acc_ref, x0_ref, dinv_ref, w_ref, r_ref, b_ref, out_ref, *, first):
    dinv = dinv_ref[...]
    if first:
        pin = dinv * (acc_ref[0] + acc_ref[1])
    else:
        pin = dinv * acc_ref[0]
    y = (jnp.dot(pin, w_ref[0], preferred_element_type=jnp.float32)
         + jnp.dot(x0_ref[...], r_ref[0], preferred_element_type=jnp.float32)
         + b_ref[0, 0])
    out_ref[...] = (dinv * _leaky(y))[None]


def _tc_stage(acc, x0, dinvb, Wt, Rt, bt, *, first):
    nb = NP // BN
    acc_spec = (pl.BlockSpec((CORES, BN, H), lambda k, i: (0, i, 0)) if first
                else pl.BlockSpec((1, BN, H), lambda k, i: (k, i, 0)))
    return pl.pallas_call(
        functools.partial(_stage_body, first=first),
        grid=(K, nb),
        in_specs=[
            acc_spec,
            pl.BlockSpec((BN, H), lambda k, i: (i, 0)),
            pl.BlockSpec((BN, H), lambda k, i: (i, 0)),
            pl.BlockSpec((1, H, H), lambda k, i: (k, 0, 0)),
            pl.BlockSpec((1, H, H), lambda k, i: (k, 0, 0)),
            pl.BlockSpec((1, 1, H), lambda k, i: (k, 0, 0)),
        ],
        out_specs=pl.BlockSpec((1, BN, H), lambda k, i: (k, i, 0)),
        out_shape=jax.ShapeDtypeStruct((K, NP, H), jnp.float32),
    )(acc, x0, dinvb, Wt, Rt, bt)


def _final_body(acc_ref, x0_ref, dinv_ref, w_ref, r_ref, b_ref,
                w3, b3, w4, b4, wr, br_ref, y_out):
    dinv = dinv_ref[...]
    x0 = x0_ref[...]
    m = jnp.zeros((BN, H), jnp.float32)
    for k in range(K):
        y = (jnp.dot(dinv * acc_ref[k], w_ref[k], preferred_element_type=jnp.float32)
             + jnp.dot(x0, r_ref[k], preferred_element_type=jnp.float32)
             + b_ref[k, 0])
        m = m + _leaky(y)
    m = m * (1.0 / K)
    h = _leaky(jnp.dot(m, w3[...], preferred_element_type=jnp.float32) + b3[...])
    h = _leaky(jnp.dot(h, w4[...], preferred_element_type=jnp.float32) + b4[...])
    y_out[...] = jnp.dot(h, wr[...], preferred_element_type=jnp.float32) + br_ref[...]


def _tc_final(acc, x0, dinvb, Wt, Rt, bt, W3, b3, W4, b4, Wr, br):
    nb = NP // BN
    return pl.pallas_call(
        _final_body,
        grid=(nb,),
        in_specs=[
            pl.BlockSpec((K, BN, H), lambda i: (0, i, 0)),
            pl.BlockSpec((BN, H), lambda i: (i, 0)),
            pl.BlockSpec((BN, H), lambda i: (i, 0)),
            pl.BlockSpec((K, H, H), lambda i: (0, 0, 0)),
            pl.BlockSpec((K, H, H), lambda i: (0, 0, 0)),
            pl.BlockSpec((K, 1, H), lambda i: (0, 0, 0)),
            pl.BlockSpec((H, H), lambda i: (0, 0)),
            pl.BlockSpec((1, H), lambda i: (0, 0)),
            pl.BlockSpec((H, H), lambda i: (0, 0)),
            pl.BlockSpec((1, H), lambda i: (0, 0)),
            pl.BlockSpec((H, D_OUT), lambda i: (0, 0)),
            pl.BlockSpec((1, D_OUT), lambda i: (0, 0)),
        ],
        out_specs=pl.BlockSpec((BN, D_OUT), lambda i: (i, 0)),
        out_shape=jax.ShapeDtypeStruct((NP, D_OUT), jnp.float32),
    )(acc, x0, dinvb, Wt, Rt, bt, W3, b3, W4, b4, Wr, br)


def kernel(x, edge_index, W1, b1, W2, b2, init_w, arma_w, root_w, arma_b,
           W3, b3, W4, b4, Wr, br):
    src = edge_index[0]
    dst = edge_index[1]

    pad16 = NC16 * C - EP16
    src16 = jnp.pad(src.reshape(TILES, EP16), ((0, 0), (0, pad16))
                    ).reshape(TILES, NC16, C)
    dst16 = jnp.pad(dst.reshape(TILES, EP16), ((0, 0), (0, pad16)),
                    constant_values=N).reshape(TILES, NC16, C)
    pad32 = NC32 * C - EP32
    src32 = jnp.pad(src.reshape(WORKERS, EP32), ((0, 0), (0, pad32))
                    ).reshape(WORKERS, NC32, C)
    dst32 = jnp.pad(dst.reshape(WORKERS, EP32), ((0, 0), (0, pad32)),
                    constant_values=N).reshape(WORKERS, NC32, C)

    xp = jnp.pad(x, ((0, NP - N), (0, 0)))
    z64 = jnp.zeros((ZR, H), jnp.float32)
    z16 = jnp.zeros((ZR, 16), jnp.float32)
    ones16 = jnp.ones((C, 16), jnp.float32)
    b1r = b1.reshape(1, H)
    b2r = b2.reshape(1, H)
    b3r = b3.reshape(1, H)
    b4r = b4.reshape(1, H)
    brr = br.reshape(1, D_OUT)

    deg2 = _sc_degree(dst32, ones16, z16)
    x0, curs, dinvb = _tc_pre(xp, W1, b1r, W2, b2r, deg2)
    acc0 = _sc_prop0(curs, src32, dst32, z64)
    cur = _tc_stage(acc0, x0, dinvb, init_w, root_w[0], arma_b[0], first=True)
    for t in range(1, T - 1):
        acc = _sc_prop(cur, src16, dst16, z64)
        cur = _tc_stage(acc, x0, dinvb, arma_w[t - 1], root_w[t], arma_b[t],
                        first=False)
    acc = _sc_prop(cur, src16, dst16, z64)
    y = _tc_final(acc, x0, dinvb, arma_w[T - 2], root_w[T - 1], arma_b[T - 1],
                  W3, b3r, W4, b4r, Wr, brr)
    return y[:N]


# speedup vs baseline: 35.9006x; 35.9006x over previous
import functools

import jax
import jax.numpy as jnp
from jax import lax
from jax.experimental import pallas as pl
from jax.experimental.pallas import tpu as pltpu
from jax.experimental.pallas import tpu_sc as plsc

N = 10000
NP = 10240
E = 320000
D_IN = 128
H = 64
K = 8
T = 4
D_OUT = 4

W2H = 2 * H
KP = K // 2

C = 128
TILES = 16
CORES = 2
WORKERS = CORES * TILES
EP16 = E // TILES
BLK = 16
NBLK = -(-EP16 // (BLK * C))
NC16 = NBLK * BLK
EP32 = E // WORKERS
NC32 = -(-EP32 // C)
ZR = NP // TILES

_mesh = plsc.VectorSubcoreMesh(core_axis_name="c", subcore_axis_name="s")


@functools.partial(
    pl.kernel,
    out_type=jax.ShapeDtypeStruct((CORES, NP, 16), jnp.float32),
    mesh=_mesh,
    scratch_types=[
        pltpu.VMEM((NC32, C), jnp.int32),
        pltpu.VMEM((C, 16), jnp.float32),
        pltpu.VMEM_SHARED((NP, 16), jnp.float32),
    ],
)
def _sc_degree(dst32, ones_hbm, z16, deg_out, idx_v, ones_v, acc_sh):
    c = lax.axis_index("c")
    s = lax.axis_index("s")
    wid = c * TILES + s
    pltpu.sync_copy(dst32.at[wid], idx_v)
    pltpu.sync_copy(ones_hbm, ones_v)
    pltpu.sync_copy(z16, acc_sh.at[pl.ds(s * ZR, ZR)])
    plsc.subcore_barrier()

    def chunk(g, carry):
        pltpu.sync_copy(ones_v, acc_sh.at[idx_v.at[g]], add=True)
        return carry

    lax.fori_loop(0, NC32, chunk, 0)
    plsc.subcore_barrier()
    pltpu.sync_copy(acc_sh.at[pl.ds(s * ZR, ZR)], deg_out.at[c, pl.ds(s * ZR, ZR)])


@functools.partial(
    pl.kernel,
    out_type=jax.ShapeDtypeStruct((CORES, NP, W2H), jnp.float32),
    mesh=_mesh,
    scratch_types=[
        pltpu.VMEM((NC32, C), jnp.int32),
        pltpu.VMEM((NC32, C), jnp.int32),
        pltpu.VMEM((C, W2H), jnp.float32),
        pltpu.VMEM_SHARED((NP, W2H), jnp.float32),
        pltpu.SemaphoreType.DMA,
    ],
)
def _sc_prop0(cur_hbm, src32, dst32, z128, out, idx_s, idx_d, rows_v, acc_sh, sem):
    c = lax.axis_index("c")
    s = lax.axis_index("s")
    wid = c * TILES + s
    pltpu.sync_copy(src32.at[wid], idx_s)
    pltpu.sync_copy(dst32.at[wid], idx_d)
    pltpu.sync_copy(z128, acc_sh.at[pl.ds(s * ZR, ZR)])
    plsc.subcore_barrier()

    def chunk(g, carry):
        pltpu.async_copy(cur_hbm.at[idx_s.at[g]], rows_v, sem).wait()
        pltpu.sync_copy(rows_v, acc_sh.at[idx_d.at[g]], add=True)
        return carry

    lax.fori_loop(0, NC32, chunk, 0)
    plsc.subcore_barrier()
    pltpu.sync_copy(acc_sh.at[pl.ds(s * ZR, ZR)], out.at[c, pl.ds(s * ZR, ZR)])


@functools.partial(
    pl.kernel,
    out_type=jax.ShapeDtypeStruct((KP, NP, W2H), jnp.float32),
    mesh=_mesh,
    scratch_types=[
        pltpu.VMEM((BLK, C), jnp.int32),
        pltpu.VMEM((BLK, C), jnp.int32),
        pltpu.VMEM((C, W2H), jnp.float32),
        pltpu.VMEM_SHARED((NP, W2H), jnp.float32),
        pltpu.SemaphoreType.DMA,
    ],
)
def _sc_prop(cur_hbm, src16, dst16, z128, out, idx_s, idx_d, rows_v, acc_sh, sem):
    c = lax.axis_index("c")
    s = lax.axis_index("s")
    for jj in range(KP // CORES):
        j = c * (KP // CORES) + jj
        pltpu.sync_copy(z128, acc_sh.at[pl.ds(s * ZR, ZR)])
        plsc.subcore_barrier()

        def blk(b, carry):
            pltpu.sync_copy(src16.at[s, b], idx_s)
            pltpu.sync_copy(dst16.at[s, b], idx_d)

            def chunk(g, carry2):
                pltpu.async_copy(cur_hbm.at[j].at[idx_s.at[g]], rows_v, sem).wait()
                pltpu.sync_copy(rows_v, acc_sh.at[idx_d.at[g]], add=True)
                return carry2

            lax.fori_loop(0, BLK, chunk, 0)
            return carry

        lax.fori_loop(0, NBLK, blk, 0)
        plsc.subcore_barrier()
        pltpu.sync_copy(acc_sh.at[pl.ds(s * ZR, ZR)], out.at[j, pl.ds(s * ZR, ZR)])


def _leaky(v):
    return jnp.where(v >= 0, v, 0.2 * v)


BN = 2048


def _pre_body(x_ref, w1, b1, w2, b2, deg_ref, x0_out, curs_out, dinv_out):
    xb = x_ref[...]
    xb = jnp.where(jnp.isnan(xb), 0.0, xb)
    h1 = _leaky(jnp.dot(xb, w1[...], preferred_element_type=jnp.float32) + b1[...])
    h2 = _leaky(jnp.dot(h1, w2[...], preferred_element_type=jnp.float32) + b2[...])
    deg = deg_ref[0, :, 0:1] + deg_ref[1, :, 0:1]
    dinv = jnp.where(deg > 0, lax.rsqrt(jnp.where(deg > 0, deg, 1.0)), 0.0)
    x0_out[...] = h2
    dinv_out[...] = jnp.broadcast_to(dinv, (BN, H))
    cs = h2 * dinv
    curs_out[...] = jnp.concatenate([cs, cs], axis=1)


def _tc_pre(xp, W1, b1, W2, b2, deg2):
    nb = NP // BN
    return pl.pallas_call(
        _pre_body,
        grid=(nb,),
        in_specs=[
            pl.BlockSpec((BN, D_IN), lambda i: (i, 0)),
            pl.BlockSpec((D_IN, H), lambda i: (0, 0)),
            pl.BlockSpec((1, H), lambda i: (0, 0)),
            pl.BlockSpec((H, H), lambda i: (0, 0)),
            pl.BlockSpec((1, H), lambda i: (0, 0)),
            pl.BlockSpec((CORES, BN, 16), lambda i: (0, i, 0)),
        ],
        out_specs=[
            pl.BlockSpec((BN, H), lambda i: (i, 0)),
            pl.BlockSpec((BN, W2H), lambda i: (i, 0)),
            pl.BlockSpec((BN, H), lambda i: (i, 0)),
        ],
        out_shape=[
            jax.ShapeDtypeStruct((NP, H), jnp.float32),
            jax.ShapeDtypeStruct((NP, W2H), jnp.float32),
            jax.ShapeDtypeStruct((NP, H), jnp.float32),
        ],
    )(xp, W1, b1, W2, b2, deg2)


def _stage_body(acc_ref, x0_ref, dinv_ref, w_ref, r_ref, b_ref, out_ref, *, first):
    dinv = dinv_ref[...]
    x0 = x0_ref[...]
    if first:
        pin0 = dinv * (acc_ref[0, :, 0:H] + acc_ref[1, :, 0:H])
        pin1 = pin0
    else:
        pin0 = dinv * acc_ref[0, :, 0:H]
        pin1 = dinv * acc_ref[0, :, H:W2H]
    y0 = (jnp.dot(pin0, w_ref[0], preferred_element_type=jnp.float32)
          + jnp.dot(x0, r_ref[0], preferred_element_type=jnp.float32)
          + b_ref[0, 0])
    y1 = (jnp.dot(pin1, w_ref[1], preferred_element_type=jnp.float32)
          + jnp.dot(x0, r_ref[1], preferred_element_type=jnp.float32)
          + b_ref[1, 0])
    out_ref[...] = jnp.concatenate([dinv * _leaky(y0), dinv * _leaky(y1)],
                                   axis=1)[None]


def _tc_stage(acc, x0, dinvb, Wt, Rt, bt, *, first):
    nb = NP // BN
    acc_spec = (pl.BlockSpec((CORES, BN, W2H), lambda j, i: (0, i, 0)) if first
                else pl.BlockSpec((1, BN, W2H), lambda j, i: (j, i, 0)))
    return pl.pallas_call(
        functools.partial(_stage_body, first=first),
        grid=(KP, nb),
        in_specs=[
            acc_spec,
            pl.BlockSpec((BN, H), lambda j, i: (i, 0)),
            pl.BlockSpec((BN, H), lambda j, i: (i, 0)),
            pl.BlockSpec((2, H, H), lambda j, i: (j, 0, 0)),
            pl.BlockSpec((2, H, H), lambda j, i: (j, 0, 0)),
            pl.BlockSpec((2, 1, H), lambda j, i: (j, 0, 0)),
        ],
        out_specs=pl.BlockSpec((1, BN, W2H), lambda j, i: (j, i, 0)),
        out_shape=jax.ShapeDtypeStruct((KP, NP, W2H), jnp.float32),
    )(acc, x0, dinvb, Wt, Rt, bt)


def _final_body(acc_ref, x0_ref, dinv_ref, w_ref, r_ref, b_ref,
                w3, b3, w4, b4, wr, br_ref, y_out):
    dinv = dinv_ref[...]
    x0 = x0_ref[...]
    m = jnp.zeros((BN, H), jnp.float32)
    for k in range(K):
        j, half = divmod(k, 2)
        pin = dinv * acc_ref[j, :, half * H:(half + 1) * H]
        y = (jnp.dot(pin, w_ref[k], preferred_element_type=jnp.float32)
             + jnp.dot(x0, r_ref[k], preferred_element_type=jnp.float32)
             + b_ref[k, 0])
        m = m + _leaky(y)
    m = m * (1.0 / K)
    h = _leaky(jnp.dot(m, w3[...], preferred_element_type=jnp.float32) + b3[...])
    h = _leaky(jnp.dot(h, w4[...], preferred_element_type=jnp.float32) + b4[...])
    y_out[...] = jnp.dot(h, wr[...], preferred_element_type=jnp.float32) + br_ref[...]


def _tc_final(acc, x0, dinvb, Wt, Rt, bt, W3, b3, W4, b4, Wr, br):
    nb = NP // BN
    return pl.pallas_call(
        _final_body,
        grid=(nb,),
        in_specs=[
            pl.BlockSpec((KP, BN, W2H), lambda i: (0, i, 0)),
            pl.BlockSpec((BN, H), lambda i: (i, 0)),
            pl.BlockSpec((BN, H), lambda i: (i, 0)),
            pl.BlockSpec((K, H, H), lambda i: (0, 0, 0)),
            pl.BlockSpec((K, H, H), lambda i: (0, 0, 0)),
            pl.BlockSpec((K, 1, H), lambda i: (0, 0, 0)),
            pl.BlockSpec((H, H), lambda i: (0, 0)),
            pl.BlockSpec((1, H), lambda i: (0, 0)),
            pl.BlockSpec((H, H), lambda i: (0, 0)),
            pl.BlockSpec((1, H), lambda i: (0, 0)),
            pl.BlockSpec((H, D_OUT), lambda i: (0, 0)),
            pl.BlockSpec((1, D_OUT), lambda i: (0, 0)),
        ],
        out_specs=pl.BlockSpec((BN, D_OUT), lambda i: (i, 0)),
        out_shape=jax.ShapeDtypeStruct((NP, D_OUT), jnp.float32),
    )(acc, x0, dinvb, Wt, Rt, bt, W3, b3, W4, b4, Wr, br)


def kernel(x, edge_index, W1, b1, W2, b2, init_w, arma_w, root_w, arma_b,
           W3, b3, W4, b4, Wr, br):
    src = edge_index[0]
    dst = edge_index[1]

    pad16 = NC16 * C - EP16
    src16 = jnp.pad(src.reshape(TILES, EP16), ((0, 0), (0, pad16))
                    ).reshape(TILES, NBLK, BLK, C)
    dst16 = jnp.pad(dst.reshape(TILES, EP16), ((0, 0), (0, pad16)),
                    constant_values=N).reshape(TILES, NBLK, BLK, C)
    pad32 = NC32 * C - EP32
    src32 = jnp.pad(src.reshape(WORKERS, EP32), ((0, 0), (0, pad32))
                    ).reshape(WORKERS, NC32, C)
    dst32 = jnp.pad(dst.reshape(WORKERS, EP32), ((0, 0), (0, pad32)),
                    constant_values=N).reshape(WORKERS, NC32, C)

    xp = jnp.pad(x, ((0, NP - N), (0, 0)))
    z128 = jnp.zeros((ZR, W2H), jnp.float32)
    z16 = jnp.zeros((ZR, 16), jnp.float32)
    ones16 = jnp.ones((C, 16), jnp.float32)
    b1r = b1.reshape(1, H)
    b2r = b2.reshape(1, H)
    b3r = b3.reshape(1, H)
    b4r = b4.reshape(1, H)
    brr = br.reshape(1, D_OUT)

    deg2 = _sc_degree(dst32, ones16, z16)
    x0, curs, dinvb = _tc_pre(xp, W1, b1r, W2, b2r, deg2)
    acc0 = _sc_prop0(curs, src32, dst32, z128)
    cur = _tc_stage(acc0, x0, dinvb, init_w, root_w[0], arma_b[0], first=True)
    for t in range(1, T - 1):
        acc = _sc_prop(cur, src16, dst16, z128)
        cur = _tc_stage(acc, x0, dinvb, arma_w[t - 1], root_w[t], arma_b[t],
                        first=False)
    acc = _sc_prop(cur, src16, dst16, z128)
    y = _tc_final(acc, x0, dinvb, arma_w[T - 2], root_w[T - 1], arma_b[T - 1],
                  W3, b3r, W4, b4r, Wr, brr)
    return y[:N]


# speedup vs baseline: 37.0322x; 1.0315x over previous
import functools

import jax
import jax.numpy as jnp
from jax import lax
from jax.experimental import pallas as pl
from jax.experimental.pallas import tpu as pltpu
from jax.experimental.pallas import tpu_sc as plsc

N = 10000
NP = 10240
E = 320000
D_IN = 128
H = 64
K = 8
T = 4
D_OUT = 4

W2H = 2 * H
KP = K // 2

C = 128
TILES = 16
CORES = 2
WORKERS = CORES * TILES
EP16 = E // TILES
BLK = 16
NBLK = -(-EP16 // (BLK * C))
NC16 = NBLK * BLK
EP32 = E // WORKERS
NBLK32 = -(-EP32 // (BLK * C))
NC32 = NBLK32 * BLK
ZR = NP // TILES

_mesh = plsc.VectorSubcoreMesh(core_axis_name="c", subcore_axis_name="s")


@functools.partial(
    pl.kernel,
    out_type=jax.ShapeDtypeStruct((CORES, NP, 16), jnp.float32),
    mesh=_mesh,
    scratch_types=[
        pltpu.VMEM((NBLK32, BLK, C), jnp.int32),
        pltpu.VMEM((C, 16), jnp.float32),
        pltpu.VMEM_SHARED((NP, 16), jnp.float32),
    ],
)
def _sc_degree(dst32, ones_hbm, z16, deg_out, idx_v, ones_v, acc_sh):
    c = lax.axis_index("c")
    s = lax.axis_index("s")
    wid = c * TILES + s
    pltpu.sync_copy(dst32.at[wid], idx_v)
    pltpu.sync_copy(ones_hbm, ones_v)
    pltpu.sync_copy(z16, acc_sh.at[pl.ds(s * ZR, ZR)])
    plsc.subcore_barrier()

    def blk(b, carry):
        def chunk(g, carry2):
            pltpu.sync_copy(ones_v, acc_sh.at[idx_v.at[b, g]], add=True)
            return carry2

        lax.fori_loop(0, BLK, chunk, 0)
        return carry

    lax.fori_loop(0, NBLK32, blk, 0)
    plsc.subcore_barrier()
    pltpu.sync_copy(acc_sh.at[pl.ds(s * ZR, ZR)], deg_out.at[c, pl.ds(s * ZR, ZR)])


@functools.partial(
    pl.kernel,
    out_type=jax.ShapeDtypeStruct((CORES, NP, W2H), jnp.float32),
    mesh=_mesh,
    scratch_types=[
        pltpu.VMEM((BLK, C), jnp.int32),
        pltpu.VMEM((BLK, C), jnp.int32),
        pltpu.VMEM((C, W2H), jnp.float32),
        pltpu.VMEM((C, W2H), jnp.float32),
        pltpu.VMEM_SHARED((NP, W2H), jnp.float32),
        pltpu.SemaphoreType.DMA,
        pltpu.SemaphoreType.DMA,
    ],
)
def _sc_prop0(cur_hbm, src32, dst32, z128, out,
              idx_s, idx_d, rows_a, rows_b, acc_sh, sem_a, sem_b):
    c = lax.axis_index("c")
    s = lax.axis_index("s")
    wid = c * TILES + s
    pltpu.sync_copy(z128, acc_sh.at[pl.ds(s * ZR, ZR)])
    plsc.subcore_barrier()

    def blk(b, carry):
        pltpu.sync_copy(src32.at[wid, b], idx_s)
        pltpu.sync_copy(dst32.at[wid, b], idx_d)

        def pair(p, carry2):
            g = 2 * p
            cp_a = pltpu.async_copy(cur_hbm.at[idx_s.at[g]], rows_a, sem_a)
            cp_b = pltpu.async_copy(cur_hbm.at[idx_s.at[g + 1]], rows_b, sem_b)
            cp_a.wait()
            pltpu.sync_copy(rows_a, acc_sh.at[idx_d.at[g]], add=True)
            cp_b.wait()
            pltpu.sync_copy(rows_b, acc_sh.at[idx_d.at[g + 1]], add=True)
            return carry2

        lax.fori_loop(0, BLK // 2, pair, 0)
        return carry

    lax.fori_loop(0, NBLK32, blk, 0)
    plsc.subcore_barrier()
    pltpu.sync_copy(acc_sh.at[pl.ds(s * ZR, ZR)], out.at[c, pl.ds(s * ZR, ZR)])


@functools.partial(
    pl.kernel,
    out_type=jax.ShapeDtypeStruct((KP, NP, W2H), jnp.float32),
    mesh=_mesh,
    scratch_types=[
        pltpu.VMEM((BLK, C), jnp.int32),
        pltpu.VMEM((BLK, C), jnp.int32),
        pltpu.VMEM((C, W2H), jnp.float32),
        pltpu.VMEM((C, W2H), jnp.float32),
        pltpu.VMEM_SHARED((NP, W2H), jnp.float32),
        pltpu.SemaphoreType.DMA,
        pltpu.SemaphoreType.DMA,
    ],
)
def _sc_prop(cur_hbm, src16, dst16, z128, out,
             idx_s, idx_d, rows_a, rows_b, acc_sh, sem_a, sem_b):
    c = lax.axis_index("c")
    s = lax.axis_index("s")
    for jj in range(KP // CORES):
        j = c * (KP // CORES) + jj
        pltpu.sync_copy(z128, acc_sh.at[pl.ds(s * ZR, ZR)])
        plsc.subcore_barrier()

        def blk(b, carry):
            pltpu.sync_copy(src16.at[s, b], idx_s)
            pltpu.sync_copy(dst16.at[s, b], idx_d)

            def pair(p, carry2):
                g = 2 * p
                cp_a = pltpu.async_copy(cur_hbm.at[j].at[idx_s.at[g]], rows_a, sem_a)
                cp_b = pltpu.async_copy(cur_hbm.at[j].at[idx_s.at[g + 1]], rows_b, sem_b)
                cp_a.wait()
                pltpu.sync_copy(rows_a, acc_sh.at[idx_d.at[g]], add=True)
                cp_b.wait()
                pltpu.sync_copy(rows_b, acc_sh.at[idx_d.at[g + 1]], add=True)
                return carry2

            lax.fori_loop(0, BLK // 2, pair, 0)
            return carry

        lax.fori_loop(0, NBLK, blk, 0)
        plsc.subcore_barrier()
        pltpu.sync_copy(acc_sh.at[pl.ds(s * ZR, ZR)], out.at[j, pl.ds(s * ZR, ZR)])


def _leaky(v):
    return jnp.where(v >= 0, v, 0.2 * v)


BN = 2048


def _pre_body(x_ref, w1, b1, w2, b2, deg_ref, x0_out, curs_out, dinv_out):
    xb = x_ref[...]
    xb = jnp.where(jnp.isnan(xb), 0.0, xb)
    h1 = _leaky(jnp.dot(xb, w1[...], preferred_element_type=jnp.float32) + b1[...])
    h2 = _leaky(jnp.dot(h1, w2[...], preferred_element_type=jnp.float32) + b2[...])
    deg = deg_ref[0, :, 0:1] + deg_ref[1, :, 0:1]
    dinv = jnp.where(deg > 0, lax.rsqrt(jnp.where(deg > 0, deg, 1.0)), 0.0)
    x0_out[...] = h2
    dinv_out[...] = jnp.broadcast_to(dinv, (BN, H))
    cs = h2 * dinv
    curs_out[...] = jnp.concatenate([cs, cs], axis=1)


def _tc_pre(xp, W1, b1, W2, b2, deg2):
    nb = NP // BN
    return pl.pallas_call(
        _pre_body,
        grid=(nb,),
        in_specs=[
            pl.BlockSpec((BN, D_IN), lambda i: (i, 0)),
            pl.BlockSpec((D_IN, H), lambda i: (0, 0)),
            pl.BlockSpec((1, H), lambda i: (0, 0)),
            pl.BlockSpec((H, H), lambda i: (0, 0)),
            pl.BlockSpec((1, H), lambda i: (0, 0)),
            pl.BlockSpec((CORES, BN, 16), lambda i: (0, i, 0)),
        ],
        out_specs=[
            pl.BlockSpec((BN, H), lambda i: (i, 0)),
            pl.BlockSpec((BN, W2H), lambda i: (i, 0)),
            pl.BlockSpec((BN, H), lambda i: (i, 0)),
        ],
        out_shape=[
            jax.ShapeDtypeStruct((NP, H), jnp.float32),
            jax.ShapeDtypeStruct((NP, W2H), jnp.float32),
            jax.ShapeDtypeStruct((NP, H), jnp.float32),
        ],
    )(xp, W1, b1, W2, b2, deg2)


def _stage_body(acc_ref, x0_ref, dinv_ref, w_ref, r_ref, b_ref, out_ref, *, first):
    dinv = dinv_ref[...]
    x0 = x0_ref[...]
    if first:
        pin0 = dinv * (acc_ref[0, :, 0:H] + acc_ref[1, :, 0:H])
        pin1 = pin0
    else:
        pin0 = dinv * acc_ref[0, :, 0:H]
        pin1 = dinv * acc_ref[0, :, H:W2H]
    y0 = (jnp.dot(pin0, w_ref[0], preferred_element_type=jnp.float32)
          + jnp.dot(x0, r_ref[0], preferred_element_type=jnp.float32)
          + b_ref[0, 0])
    y1 = (jnp.dot(pin1, w_ref[1], preferred_element_type=jnp.float32)
          + jnp.dot(x0, r_ref[1], preferred_element_type=jnp.float32)
          + b_ref[1, 0])
    out_ref[...] = jnp.concatenate([dinv * _leaky(y0), dinv * _leaky(y1)],
                                   axis=1)[None]


def _tc_stage(acc, x0, dinvb, Wt, Rt, bt, *, first):
    nb = NP // BN
    acc_spec = (pl.BlockSpec((CORES, BN, W2H), lambda j, i: (0, i, 0)) if first
                else pl.BlockSpec((1, BN, W2H), lambda j, i: (j, i, 0)))
    return pl.pallas_call(
        functools.partial(_stage_body, first=first),
        grid=(KP, nb),
        in_specs=[
            acc_spec,
            pl.BlockSpec((BN, H), lambda j, i: (i, 0)),
            pl.BlockSpec((BN, H), lambda j, i: (i, 0)),
            pl.BlockSpec((2, H, H), lambda j, i: (j, 0, 0)),
            pl.BlockSpec((2, H, H), lambda j, i: (j, 0, 0)),
            pl.BlockSpec((2, 1, H), lambda j, i: (j, 0, 0)),
        ],
        out_specs=pl.BlockSpec((1, BN, W2H), lambda j, i: (j, i, 0)),
        out_shape=jax.ShapeDtypeStruct((KP, NP, W2H), jnp.float32),
    )(acc, x0, dinvb, Wt, Rt, bt)


def _final_body(acc_ref, x0_ref, dinv_ref, w_ref, r_ref, b_ref,
                w3, b3, w4, b4, wr, br_ref, y_out):
    dinv = dinv_ref[...]
    x0 = x0_ref[...]
    m = jnp.zeros((BN, H), jnp.float32)
    for k in range(K):
        j, half = divmod(k, 2)
        pin = dinv * acc_ref[j, :, half * H:(half + 1) * H]
        y = (jnp.dot(pin, w_ref[k], preferred_element_type=jnp.float32)
             + jnp.dot(x0, r_ref[k], preferred_element_type=jnp.float32)
             + b_ref[k, 0])
        m = m + _leaky(y)
    m = m * (1.0 / K)
    h = _leaky(jnp.dot(m, w3[...], preferred_element_type=jnp.float32) + b3[...])
    h = _leaky(jnp.dot(h, w4[...], preferred_element_type=jnp.float32) + b4[...])
    y_out[...] = jnp.dot(h, wr[...], preferred_element_type=jnp.float32) + br_ref[...]


def _tc_final(acc, x0, dinvb, Wt, Rt, bt, W3, b3, W4, b4, Wr, br):
    nb = NP // BN
    return pl.pallas_call(
        _final_body,
        grid=(nb,),
        in_specs=[
            pl.BlockSpec((KP, BN, W2H), lambda i: (0, i, 0)),
            pl.BlockSpec((BN, H), lambda i: (i, 0)),
            pl.BlockSpec((BN, H), lambda i: (i, 0)),
            pl.BlockSpec((K, H, H), lambda i: (0, 0, 0)),
            pl.BlockSpec((K, H, H), lambda i: (0, 0, 0)),
            pl.BlockSpec((K, 1, H), lambda i: (0, 0, 0)),
            pl.BlockSpec((H, H), lambda i: (0, 0)),
            pl.BlockSpec((1, H), lambda i: (0, 0)),
            pl.BlockSpec((H, H), lambda i: (0, 0)),
            pl.BlockSpec((1, H), lambda i: (0, 0)),
            pl.BlockSpec((H, D_OUT), lambda i: (0, 0)),
            pl.BlockSpec((1, D_OUT), lambda i: (0, 0)),
        ],
        out_specs=pl.BlockSpec((BN, D_OUT), lambda i: (i, 0)),
        out_shape=jax.ShapeDtypeStruct((NP, D_OUT), jnp.float32),
    )(acc, x0, dinvb, Wt, Rt, bt, W3, b3, W4, b4, Wr, br)


def kernel(x, edge_index, W1, b1, W2, b2, init_w, arma_w, root_w, arma_b,
           W3, b3, W4, b4, Wr, br):
    src = edge_index[0]
    dst = edge_index[1]

    pad16 = NC16 * C - EP16
    src16 = jnp.pad(src.reshape(TILES, EP16), ((0, 0), (0, pad16))
                    ).reshape(TILES, NBLK, BLK, C)
    dst16 = jnp.pad(dst.reshape(TILES, EP16), ((0, 0), (0, pad16)),
                    constant_values=N).reshape(TILES, NBLK, BLK, C)
    pad32 = NC32 * C - EP32
    src32 = jnp.pad(src.reshape(WORKERS, EP32), ((0, 0), (0, pad32))
                    ).reshape(WORKERS, NBLK32, BLK, C)
    dst32 = jnp.pad(dst.reshape(WORKERS, EP32), ((0, 0), (0, pad32)),
                    constant_values=N).reshape(WORKERS, NBLK32, BLK, C)

    xp = jnp.pad(x, ((0, NP - N), (0, 0)))
    z128 = jnp.zeros((ZR, W2H), jnp.float32)
    z16 = jnp.zeros((ZR, 16), jnp.float32)
    ones16 = jnp.ones((C, 16), jnp.float32)
    b1r = b1.reshape(1, H)
    b2r = b2.reshape(1, H)
    b3r = b3.reshape(1, H)
    b4r = b4.reshape(1, H)
    brr = br.reshape(1, D_OUT)

    deg2 = _sc_degree(dst32, ones16, z16)
    x0, curs, dinvb = _tc_pre(xp, W1, b1r, W2, b2r, deg2)
    acc0 = _sc_prop0(curs, src32, dst32, z128)
    cur = _tc_stage(acc0, x0, dinvb, init_w, root_w[0], arma_b[0], first=True)
    for t in range(1, T - 1):
        acc = _sc_prop(cur, src16, dst16, z128)
        cur = _tc_stage(acc, x0, dinvb, arma_w[t - 1], root_w[t], arma_b[t],
                        first=False)
    acc = _sc_prop(cur, src16, dst16, z128)
    y = _tc_final(acc, x0, dinvb, arma_w[T - 2], root_w[T - 1], arma_b[T - 1],
                  W3, b3r, W4, b4r, Wr, brr)
    return y[:N]


# speedup vs baseline: 41.4140x; 1.1183x over previous
import functools

import jax
import jax.numpy as jnp
from jax import lax
from jax.experimental import pallas as pl
from jax.experimental.pallas import tpu as pltpu
from jax.experimental.pallas import tpu_sc as plsc

N = 10000
NP = 10240
E = 320000
D_IN = 128
H = 64
K = 8
T = 4
D_OUT = 4

W2H = 2 * H
KP = K // 2

C = 128
TILES = 16
CORES = 2
WORKERS = CORES * TILES
EP16 = E // TILES
BLK = 16
NBLK = -(-EP16 // (BLK * C))
NC16 = NBLK * BLK
EP32 = E // WORKERS
NBLK32 = -(-EP32 // (BLK * C))
NC32 = NBLK32 * BLK
ZR = NP // TILES

_mesh = plsc.VectorSubcoreMesh(core_axis_name="c", subcore_axis_name="s")


@functools.partial(
    pl.kernel,
    out_type=jax.ShapeDtypeStruct((CORES, NP, 16), jnp.float32),
    mesh=_mesh,
    scratch_types=[
        pltpu.VMEM((NBLK32, BLK, C), jnp.int32),
        pltpu.VMEM((C, 16), jnp.float32),
        pltpu.VMEM_SHARED((NP, 16), jnp.float32),
    ],
)
def _sc_degree(dst32, ones_hbm, z16, deg_out, idx_v, ones_v, acc_sh):
    c = lax.axis_index("c")
    s = lax.axis_index("s")
    wid = c * TILES + s
    pltpu.sync_copy(dst32.at[wid], idx_v)
    pltpu.sync_copy(ones_hbm, ones_v)
    pltpu.sync_copy(z16, acc_sh.at[pl.ds(s * ZR, ZR)])
    plsc.subcore_barrier()

    def blk(b, carry):
        def chunk(g, carry2):
            pltpu.sync_copy(ones_v, acc_sh.at[idx_v.at[b, g]], add=True)
            return carry2

        lax.fori_loop(0, BLK, chunk, 0)
        return carry

    lax.fori_loop(0, NBLK32, blk, 0)
    plsc.subcore_barrier()
    pltpu.sync_copy(acc_sh.at[pl.ds(s * ZR, ZR)], deg_out.at[c, pl.ds(s * ZR, ZR)])


@functools.partial(
    pl.kernel,
    out_type=jax.ShapeDtypeStruct((CORES, NP, W2H), jnp.float32),
    mesh=_mesh,
    scratch_types=[
        pltpu.VMEM((BLK, C), jnp.int32),
        pltpu.VMEM((BLK, C), jnp.int32),
        pltpu.VMEM((C, W2H), jnp.float32),
        pltpu.VMEM((C, W2H), jnp.float32),
        pltpu.VMEM_SHARED((NP, W2H), jnp.float32),
        pltpu.SemaphoreType.DMA,
        pltpu.SemaphoreType.DMA,
    ],
)
def _sc_prop0(cur_hbm, src32, dst32, z128, out,
              idx_s, idx_d, rows_a, rows_b, acc_sh, sem_a, sem_b):
    c = lax.axis_index("c")
    s = lax.axis_index("s")
    wid = c * TILES + s
    pltpu.sync_copy(z128, acc_sh.at[pl.ds(s * ZR, ZR)])
    plsc.subcore_barrier()

    def blk(b, carry):
        pltpu.sync_copy(src32.at[wid, b], idx_s)
        pltpu.sync_copy(dst32.at[wid, b], idx_d)

        def pair(p, carry2):
            g = 2 * p
            cp_a = pltpu.async_copy(cur_hbm.at[idx_s.at[g]], rows_a, sem_a)
            cp_b = pltpu.async_copy(cur_hbm.at[idx_s.at[g + 1]], rows_b, sem_b)
            cp_a.wait()
            pltpu.sync_copy(rows_a, acc_sh.at[idx_d.at[g]], add=True)
            cp_b.wait()
            pltpu.sync_copy(rows_b, acc_sh.at[idx_d.at[g + 1]], add=True)
            return carry2

        lax.fori_loop(0, BLK // 2, pair, 0)
        return carry

    lax.fori_loop(0, NBLK32, blk, 0)
    plsc.subcore_barrier()
    pltpu.sync_copy(acc_sh.at[pl.ds(s * ZR, ZR)], out.at[c, pl.ds(s * ZR, ZR)])


@functools.partial(
    pl.kernel,
    out_type=jax.ShapeDtypeStruct((KP, NP, W2H), jnp.float32),
    mesh=_mesh,
    scratch_types=[
        pltpu.VMEM((BLK, C), jnp.int32),
        pltpu.VMEM((BLK, C), jnp.int32),
        pltpu.VMEM((C, W2H), jnp.float32),
        pltpu.VMEM((C, W2H), jnp.float32),
        pltpu.VMEM_SHARED((NP, W2H), jnp.float32),
        pltpu.SemaphoreType.DMA,
        pltpu.SemaphoreType.DMA,
    ],
)
def _sc_prop(cur_hbm, src16, dst16, z128, out,
             idx_s, idx_d, rows_a, rows_b, acc_sh, sem_a, sem_b):
    c = lax.axis_index("c")
    s = lax.axis_index("s")
    for jj in range(KP // CORES):
        j = c * (KP // CORES) + jj
        pltpu.sync_copy(z128, acc_sh.at[pl.ds(s * ZR, ZR)])
        plsc.subcore_barrier()

        def blk(b, carry):
            pltpu.sync_copy(src16.at[s, b], idx_s)
            pltpu.sync_copy(dst16.at[s, b], idx_d)
            cur2 = cur_hbm.at[j]
            pltpu.async_copy(cur2.at[idx_s.at[0]], rows_a, sem_a)
            pltpu.async_copy(cur2.at[idx_s.at[1]], rows_b, sem_b)

            def pair(p, carry2):
                g = 2 * p
                pltpu.make_async_copy(cur2.at[idx_s.at[g - 2]], rows_a,
                                      sem_a).wait()
                pltpu.sync_copy(rows_a, acc_sh.at[idx_d.at[g - 2]], add=True)
                pltpu.async_copy(cur2.at[idx_s.at[g]], rows_a, sem_a)
                pltpu.make_async_copy(cur2.at[idx_s.at[g - 1]], rows_b,
                                      sem_b).wait()
                pltpu.sync_copy(rows_b, acc_sh.at[idx_d.at[g - 1]], add=True)
                pltpu.async_copy(cur2.at[idx_s.at[g + 1]], rows_b, sem_b)
                return carry2

            lax.fori_loop(1, BLK // 2, pair, 0)
            pltpu.make_async_copy(cur2.at[idx_s.at[BLK - 2]], rows_a,
                                  sem_a).wait()
            pltpu.sync_copy(rows_a, acc_sh.at[idx_d.at[BLK - 2]], add=True)
            pltpu.make_async_copy(cur2.at[idx_s.at[BLK - 1]], rows_b,
                                  sem_b).wait()
            pltpu.sync_copy(rows_b, acc_sh.at[idx_d.at[BLK - 1]], add=True)
            return carry

        lax.fori_loop(0, NBLK, blk, 0)
        plsc.subcore_barrier()
        pltpu.sync_copy(acc_sh.at[pl.ds(s * ZR, ZR)], out.at[j, pl.ds(s * ZR, ZR)])


def _leaky(v):
    return jnp.where(v >= 0, v, 0.2 * v)


BN = 2048


def _pre_body(x_ref, w1, b1, w2, b2, deg_ref, x0_out, curs_out, dinv_out):
    xb = x_ref[...]
    xb = jnp.where(jnp.isnan(xb), 0.0, xb)
    h1 = _leaky(jnp.dot(xb, w1[...], preferred_element_type=jnp.float32) + b1[...])
    h2 = _leaky(jnp.dot(h1, w2[...], preferred_element_type=jnp.float32) + b2[...])
    deg = deg_ref[0, :, 0:1] + deg_ref[1, :, 0:1]
    dinv = jnp.where(deg > 0, lax.rsqrt(jnp.where(deg > 0, deg, 1.0)), 0.0)
    x0_out[...] = h2
    dinv_out[...] = jnp.broadcast_to(dinv, (BN, H))
    cs = h2 * dinv
    curs_out[...] = jnp.concatenate([cs, cs], axis=1)


def _tc_pre(xp, W1, b1, W2, b2, deg2):
    nb = NP // BN
    return pl.pallas_call(
        _pre_body,
        grid=(nb,),
        in_specs=[
            pl.BlockSpec((BN, D_IN), lambda i: (i, 0)),
            pl.BlockSpec((D_IN, H), lambda i: (0, 0)),
            pl.BlockSpec((1, H), lambda i: (0, 0)),
            pl.BlockSpec((H, H), lambda i: (0, 0)),
            pl.BlockSpec((1, H), lambda i: (0, 0)),
            pl.BlockSpec((CORES, BN, 16), lambda i: (0, i, 0)),
        ],
        out_specs=[
            pl.BlockSpec((BN, H), lambda i: (i, 0)),
            pl.BlockSpec((BN, W2H), lambda i: (i, 0)),
            pl.BlockSpec((BN, H), lambda i: (i, 0)),
        ],
        out_shape=[
            jax.ShapeDtypeStruct((NP, H), jnp.float32),
            jax.ShapeDtypeStruct((NP, W2H), jnp.float32),
            jax.ShapeDtypeStruct((NP, H), jnp.float32),
        ],
    )(xp, W1, b1, W2, b2, deg2)


def _stage_body(acc_ref, x0_ref, dinv_ref, w_ref, r_ref, b_ref, out_ref, *, first):
    dinv = dinv_ref[...]
    x0 = x0_ref[...]
    if first:
        pin0 = dinv * (acc_ref[0, :, 0:H] + acc_ref[1, :, 0:H])
        pin1 = pin0
    else:
        pin0 = dinv * acc_ref[0, :, 0:H]
        pin1 = dinv * acc_ref[0, :, H:W2H]
    y0 = (jnp.dot(pin0, w_ref[0], preferred_element_type=jnp.float32)
          + jnp.dot(x0, r_ref[0], preferred_element_type=jnp.float32)
          + b_ref[0, 0])
    y1 = (jnp.dot(pin1, w_ref[1], preferred_element_type=jnp.float32)
          + jnp.dot(x0, r_ref[1], preferred_element_type=jnp.float32)
          + b_ref[1, 0])
    out_ref[...] = jnp.concatenate([dinv * _leaky(y0), dinv * _leaky(y1)],
                                   axis=1)[None]


def _tc_stage(acc, x0, dinvb, Wt, Rt, bt, *, first):
    nb = NP // BN
    acc_spec = (pl.BlockSpec((CORES, BN, W2H), lambda j, i: (0, i, 0)) if first
                else pl.BlockSpec((1, BN, W2H), lambda j, i: (j, i, 0)))
    return pl.pallas_call(
        functools.partial(_stage_body, first=first),
        grid=(KP, nb),
        in_specs=[
            acc_spec,
            pl.BlockSpec((BN, H), lambda j, i: (i, 0)),
            pl.BlockSpec((BN, H), lambda j, i: (i, 0)),
            pl.BlockSpec((2, H, H), lambda j, i: (j, 0, 0)),
            pl.BlockSpec((2, H, H), lambda j, i: (j, 0, 0)),
            pl.BlockSpec((2, 1, H), lambda j, i: (j, 0, 0)),
        ],
        out_specs=pl.BlockSpec((1, BN, W2H), lambda j, i: (j, i, 0)),
        out_shape=jax.ShapeDtypeStruct((KP, NP, W2H), jnp.float32),
    )(acc, x0, dinvb, Wt, Rt, bt)


def _final_body(acc_ref, x0_ref, dinv_ref, w_ref, r_ref, b_ref,
                w3, b3, w4, b4, wr, br_ref, y_out):
    dinv = dinv_ref[...]
    x0 = x0_ref[...]
    m = jnp.zeros((BN, H), jnp.float32)
    for k in range(K):
        j, half = divmod(k, 2)
        pin = dinv * acc_ref[j, :, half * H:(half + 1) * H]
        y = (jnp.dot(pin, w_ref[k], preferred_element_type=jnp.float32)
             + jnp.dot(x0, r_ref[k], preferred_element_type=jnp.float32)
             + b_ref[k, 0])
        m = m + _leaky(y)
    m = m * (1.0 / K)
    h = _leaky(jnp.dot(m, w3[...], preferred_element_type=jnp.float32) + b3[...])
    h = _leaky(jnp.dot(h, w4[...], preferred_element_type=jnp.float32) + b4[...])
    y_out[...] = jnp.dot(h, wr[...], preferred_element_type=jnp.float32) + br_ref[...]


def _tc_final(acc, x0, dinvb, Wt, Rt, bt, W3, b3, W4, b4, Wr, br):
    nb = NP // BN
    return pl.pallas_call(
        _final_body,
        grid=(nb,),
        in_specs=[
            pl.BlockSpec((KP, BN, W2H), lambda i: (0, i, 0)),
            pl.BlockSpec((BN, H), lambda i: (i, 0)),
            pl.BlockSpec((BN, H), lambda i: (i, 0)),
            pl.BlockSpec((K, H, H), lambda i: (0, 0, 0)),
            pl.BlockSpec((K, H, H), lambda i: (0, 0, 0)),
            pl.BlockSpec((K, 1, H), lambda i: (0, 0, 0)),
            pl.BlockSpec((H, H), lambda i: (0, 0)),
            pl.BlockSpec((1, H), lambda i: (0, 0)),
            pl.BlockSpec((H, H), lambda i: (0, 0)),
            pl.BlockSpec((1, H), lambda i: (0, 0)),
            pl.BlockSpec((H, D_OUT), lambda i: (0, 0)),
            pl.BlockSpec((1, D_OUT), lambda i: (0, 0)),
        ],
        out_specs=pl.BlockSpec((BN, D_OUT), lambda i: (i, 0)),
        out_shape=jax.ShapeDtypeStruct((NP, D_OUT), jnp.float32),
    )(acc, x0, dinvb, Wt, Rt, bt, W3, b3, W4, b4, Wr, br)


def kernel(x, edge_index, W1, b1, W2, b2, init_w, arma_w, root_w, arma_b,
           W3, b3, W4, b4, Wr, br):
    src = edge_index[0]
    dst = edge_index[1]

    pad16 = NC16 * C - EP16
    src16 = jnp.pad(src.reshape(TILES, EP16), ((0, 0), (0, pad16))
                    ).reshape(TILES, NBLK, BLK, C)
    dst16 = jnp.pad(dst.reshape(TILES, EP16), ((0, 0), (0, pad16)),
                    constant_values=N).reshape(TILES, NBLK, BLK, C)
    pad32 = NC32 * C - EP32
    src32 = jnp.pad(src.reshape(WORKERS, EP32), ((0, 0), (0, pad32))
                    ).reshape(WORKERS, NBLK32, BLK, C)
    dst32 = jnp.pad(dst.reshape(WORKERS, EP32), ((0, 0), (0, pad32)),
                    constant_values=N).reshape(WORKERS, NBLK32, BLK, C)

    xp = jnp.pad(x, ((0, NP - N), (0, 0)))
    z128 = jnp.zeros((ZR, W2H), jnp.float32)
    z16 = jnp.zeros((ZR, 16), jnp.float32)
    ones16 = jnp.ones((C, 16), jnp.float32)
    b1r = b1.reshape(1, H)
    b2r = b2.reshape(1, H)
    b3r = b3.reshape(1, H)
    b4r = b4.reshape(1, H)
    brr = br.reshape(1, D_OUT)

    deg2 = _sc_degree(dst32, ones16, z16)
    x0, curs, dinvb = _tc_pre(xp, W1, b1r, W2, b2r, deg2)
    acc0 = _sc_prop0(curs, src32, dst32, z128)
    cur = _tc_stage(acc0, x0, dinvb, init_w, root_w[0], arma_b[0], first=True)
    for t in range(1, T - 1):
        acc = _sc_prop(cur, src16, dst16, z128)
        cur = _tc_stage(acc, x0, dinvb, arma_w[t - 1], root_w[t], arma_b[t],
                        first=False)
    acc = _sc_prop(cur, src16, dst16, z128)
    y = _tc_final(acc, x0, dinvb, arma_w[T - 2], root_w[T - 1], arma_b[T - 1],
                  W3, b3r, W4, b4r, Wr, brr)
    return y[:N]


# speedup vs baseline: 42.7963x; 1.0334x over previous
import functools

import jax
import jax.numpy as jnp
from jax import lax
from jax.experimental import pallas as pl
from jax.experimental.pallas import tpu as pltpu
from jax.experimental.pallas import tpu_sc as plsc

N = 10000
NP = 10240
E = 320000
D_IN = 128
H = 64
K = 8
T = 4
D_OUT = 4

W2H = 2 * H
KP = K // 2

C = 128
TILES = 16
CORES = 2
WORKERS = CORES * TILES
EP16 = E // TILES
BLK = 16
BLK2 = 32
NBLK = -(-EP16 // (BLK2 * C))
NC16 = NBLK * BLK2
EP32 = E // WORKERS
NBLK32 = -(-EP32 // (BLK * C))
NC32 = NBLK32 * BLK
ZR = NP // TILES

_mesh = plsc.VectorSubcoreMesh(core_axis_name="c", subcore_axis_name="s")


@functools.partial(
    pl.kernel,
    out_type=jax.ShapeDtypeStruct((CORES, NP, 16), jnp.float32),
    mesh=_mesh,
    scratch_types=[
        pltpu.VMEM((NBLK32, BLK, C), jnp.int32),
        pltpu.VMEM((C, 16), jnp.float32),
        pltpu.VMEM_SHARED((NP, 16), jnp.float32),
    ],
)
def _sc_degree(dst32, ones_hbm, z16, deg_out, idx_v, ones_v, acc_sh):
    c = lax.axis_index("c")
    s = lax.axis_index("s")
    wid = c * TILES + s
    pltpu.sync_copy(dst32.at[wid], idx_v)
    pltpu.sync_copy(ones_hbm, ones_v)
    pltpu.sync_copy(z16, acc_sh.at[pl.ds(s * ZR, ZR)])
    plsc.subcore_barrier()

    def blk(b, carry):
        def chunk(g, carry2):
            pltpu.sync_copy(ones_v, acc_sh.at[idx_v.at[b, g]], add=True)
            return carry2

        lax.fori_loop(0, BLK, chunk, 0)
        return carry

    lax.fori_loop(0, NBLK32, blk, 0)
    plsc.subcore_barrier()
    pltpu.sync_copy(acc_sh.at[pl.ds(s * ZR, ZR)], deg_out.at[c, pl.ds(s * ZR, ZR)])


@functools.partial(
    pl.kernel,
    out_type=jax.ShapeDtypeStruct((CORES, NP, W2H), jnp.float32),
    mesh=_mesh,
    scratch_types=[
        pltpu.VMEM((BLK, C), jnp.int32),
        pltpu.VMEM((BLK, C), jnp.int32),
        pltpu.VMEM((C, W2H), jnp.float32),
        pltpu.VMEM((C, W2H), jnp.float32),
        pltpu.VMEM_SHARED((NP, W2H), jnp.float32),
        pltpu.SemaphoreType.DMA,
        pltpu.SemaphoreType.DMA,
    ],
)
def _sc_prop0(cur_hbm, src32, dst32, z128, out,
              idx_s, idx_d, rows_a, rows_b, acc_sh, sem_a, sem_b):
    c = lax.axis_index("c")
    s = lax.axis_index("s")
    wid = c * TILES + s
    pltpu.sync_copy(z128, acc_sh.at[pl.ds(s * ZR, ZR)])
    plsc.subcore_barrier()

    def blk(b, carry):
        pltpu.sync_copy(src32.at[wid, b], idx_s)
        pltpu.sync_copy(dst32.at[wid, b], idx_d)
        pltpu.async_copy(cur_hbm.at[idx_s.at[0]], rows_a, sem_a)
        pltpu.async_copy(cur_hbm.at[idx_s.at[1]], rows_b, sem_b)

        def pair(p, carry2):
            g = 2 * p
            pltpu.make_async_copy(cur_hbm.at[idx_s.at[g - 2]], rows_a,
                                  sem_a).wait()
            pltpu.sync_copy(rows_a, acc_sh.at[idx_d.at[g - 2]], add=True)
            pltpu.async_copy(cur_hbm.at[idx_s.at[g]], rows_a, sem_a)
            pltpu.make_async_copy(cur_hbm.at[idx_s.at[g - 1]], rows_b,
                                  sem_b).wait()
            pltpu.sync_copy(rows_b, acc_sh.at[idx_d.at[g - 1]], add=True)
            pltpu.async_copy(cur_hbm.at[idx_s.at[g + 1]], rows_b, sem_b)
            return carry2

        lax.fori_loop(1, BLK // 2, pair, 0)
        pltpu.make_async_copy(cur_hbm.at[idx_s.at[BLK - 2]], rows_a,
                              sem_a).wait()
        pltpu.sync_copy(rows_a, acc_sh.at[idx_d.at[BLK - 2]], add=True)
        pltpu.make_async_copy(cur_hbm.at[idx_s.at[BLK - 1]], rows_b,
                              sem_b).wait()
        pltpu.sync_copy(rows_b, acc_sh.at[idx_d.at[BLK - 1]], add=True)
        return carry

    lax.fori_loop(0, NBLK32, blk, 0)
    plsc.subcore_barrier()
    pltpu.sync_copy(acc_sh.at[pl.ds(s * ZR, ZR)], out.at[c, pl.ds(s * ZR, ZR)])


@functools.partial(
    pl.kernel,
    out_type=jax.ShapeDtypeStruct((KP, NP, W2H), jnp.float32),
    mesh=_mesh,
    scratch_types=[
        pltpu.VMEM((BLK2, C), jnp.int32),
        pltpu.VMEM((BLK2, C), jnp.int32),
        pltpu.VMEM((C, W2H), jnp.float32),
        pltpu.VMEM((C, W2H), jnp.float32),
        pltpu.VMEM_SHARED((NP, W2H), jnp.float32),
        pltpu.SemaphoreType.DMA,
        pltpu.SemaphoreType.DMA,
    ],
)
def _sc_prop(cur_hbm, src16, dst16, z128, out,
             idx_s, idx_d, rows_a, rows_b, acc_sh, sem_a, sem_b):
    c = lax.axis_index("c")
    s = lax.axis_index("s")
    for jj in range(KP // CORES):
        j = c * (KP // CORES) + jj
        pltpu.sync_copy(z128, acc_sh.at[pl.ds(s * ZR, ZR)])
        plsc.subcore_barrier()

        def blk(b, carry):
            pltpu.sync_copy(src16.at[s, b], idx_s)
            pltpu.sync_copy(dst16.at[s, b], idx_d)
            cur2 = cur_hbm.at[j]
            pltpu.async_copy(cur2.at[idx_s.at[0]], rows_a, sem_a)
            pltpu.async_copy(cur2.at[idx_s.at[1]], rows_b, sem_b)

            def pair(p, carry2):
                g = 2 * p
                pltpu.make_async_copy(cur2.at[idx_s.at[g - 2]], rows_a,
                                      sem_a).wait()
                pltpu.sync_copy(rows_a, acc_sh.at[idx_d.at[g - 2]], add=True)
                pltpu.async_copy(cur2.at[idx_s.at[g]], rows_a, sem_a)
                pltpu.make_async_copy(cur2.at[idx_s.at[g - 1]], rows_b,
                                      sem_b).wait()
                pltpu.sync_copy(rows_b, acc_sh.at[idx_d.at[g - 1]], add=True)
                pltpu.async_copy(cur2.at[idx_s.at[g + 1]], rows_b, sem_b)
                return carry2

            lax.fori_loop(1, BLK2 // 2, pair, 0)
            pltpu.make_async_copy(cur2.at[idx_s.at[BLK2 - 2]], rows_a,
                                  sem_a).wait()
            pltpu.sync_copy(rows_a, acc_sh.at[idx_d.at[BLK2 - 2]], add=True)
            pltpu.make_async_copy(cur2.at[idx_s.at[BLK2 - 1]], rows_b,
                                  sem_b).wait()
            pltpu.sync_copy(rows_b, acc_sh.at[idx_d.at[BLK2 - 1]], add=True)
            return carry

        lax.fori_loop(0, NBLK, blk, 0)
        plsc.subcore_barrier()
        pltpu.sync_copy(acc_sh.at[pl.ds(s * ZR, ZR)], out.at[j, pl.ds(s * ZR, ZR)])


def _leaky(v):
    return jnp.where(v >= 0, v, 0.2 * v)


BN = 2048


def _pre_body(x_ref, w1, b1, w2, b2, deg_ref, x0_out, curs_out, dinv_out):
    xb = x_ref[...]
    xb = jnp.where(jnp.isnan(xb), 0.0, xb)
    h1 = _leaky(jnp.dot(xb, w1[...], preferred_element_type=jnp.float32) + b1[...])
    h2 = _leaky(jnp.dot(h1, w2[...], preferred_element_type=jnp.float32) + b2[...])
    deg = deg_ref[0, :, 0:1] + deg_ref[1, :, 0:1]
    dinv = jnp.where(deg > 0, lax.rsqrt(jnp.where(deg > 0, deg, 1.0)), 0.0)
    x0_out[...] = h2
    dinv_out[...] = jnp.broadcast_to(dinv, (BN, H))
    cs = h2 * dinv
    curs_out[...] = jnp.concatenate([cs, cs], axis=1)


def _tc_pre(xp, W1, b1, W2, b2, deg2):
    nb = NP // BN
    return pl.pallas_call(
        _pre_body,
        grid=(nb,),
        in_specs=[
            pl.BlockSpec((BN, D_IN), lambda i: (i, 0)),
            pl.BlockSpec((D_IN, H), lambda i: (0, 0)),
            pl.BlockSpec((1, H), lambda i: (0, 0)),
            pl.BlockSpec((H, H), lambda i: (0, 0)),
            pl.BlockSpec((1, H), lambda i: (0, 0)),
            pl.BlockSpec((CORES, BN, 16), lambda i: (0, i, 0)),
        ],
        out_specs=[
            pl.BlockSpec((BN, H), lambda i: (i, 0)),
            pl.BlockSpec((BN, W2H), lambda i: (i, 0)),
            pl.BlockSpec((BN, H), lambda i: (i, 0)),
        ],
        out_shape=[
            jax.ShapeDtypeStruct((NP, H), jnp.float32),
            jax.ShapeDtypeStruct((NP, W2H), jnp.float32),
            jax.ShapeDtypeStruct((NP, H), jnp.float32),
        ],
    )(xp, W1, b1, W2, b2, deg2)


def _stage_body(acc_ref, x0_ref, dinv_ref, w_ref, r_ref, b_ref, out_ref, *, first):
    dinv = dinv_ref[...]
    x0 = x0_ref[...]
    if first:
        pin0 = dinv * (acc_ref[0, :, 0:H] + acc_ref[1, :, 0:H])
        pin1 = pin0
    else:
        pin0 = dinv * acc_ref[0, :, 0:H]
        pin1 = dinv * acc_ref[0, :, H:W2H]
    y0 = (jnp.dot(pin0, w_ref[0], preferred_element_type=jnp.float32)
          + jnp.dot(x0, r_ref[0], preferred_element_type=jnp.float32)
          + b_ref[0, 0])
    y1 = (jnp.dot(pin1, w_ref[1], preferred_element_type=jnp.float32)
          + jnp.dot(x0, r_ref[1], preferred_element_type=jnp.float32)
          + b_ref[1, 0])
    out_ref[...] = jnp.concatenate([dinv * _leaky(y0), dinv * _leaky(y1)],
                                   axis=1)[None]


def _tc_stage(acc, x0, dinvb, Wt, Rt, bt, *, first):
    nb = NP // BN
    acc_spec = (pl.BlockSpec((CORES, BN, W2H), lambda j, i: (0, i, 0)) if first
                else pl.BlockSpec((1, BN, W2H), lambda j, i: (j, i, 0)))
    return pl.pallas_call(
        functools.partial(_stage_body, first=first),
        grid=(KP, nb),
        in_specs=[
            acc_spec,
            pl.BlockSpec((BN, H), lambda j, i: (i, 0)),
            pl.BlockSpec((BN, H), lambda j, i: (i, 0)),
            pl.BlockSpec((2, H, H), lambda j, i: (j, 0, 0)),
            pl.BlockSpec((2, H, H), lambda j, i: (j, 0, 0)),
            pl.BlockSpec((2, 1, H), lambda j, i: (j, 0, 0)),
        ],
        out_specs=pl.BlockSpec((1, BN, W2H), lambda j, i: (j, i, 0)),
        out_shape=jax.ShapeDtypeStruct((KP, NP, W2H), jnp.float32),
    )(acc, x0, dinvb, Wt, Rt, bt)


def _final_body(acc_ref, x0_ref, dinv_ref, w_ref, r_ref, b_ref,
                w3, b3, w4, b4, wr, br_ref, y_out):
    dinv = dinv_ref[...]
    x0 = x0_ref[...]
    m = jnp.zeros((BN, H), jnp.float32)
    for k in range(K):
        j, half = divmod(k, 2)
        pin = dinv * acc_ref[j, :, half * H:(half + 1) * H]
        y = (jnp.dot(pin, w_ref[k], preferred_element_type=jnp.float32)
             + jnp.dot(x0, r_ref[k], preferred_element_type=jnp.float32)
             + b_ref[k, 0])
        m = m + _leaky(y)
    m = m * (1.0 / K)
    h = _leaky(jnp.dot(m, w3[...], preferred_element_type=jnp.float32) + b3[...])
    h = _leaky(jnp.dot(h, w4[...], preferred_element_type=jnp.float32) + b4[...])
    y_out[...] = jnp.dot(h, wr[...], preferred_element_type=jnp.float32) + br_ref[...]


def _tc_final(acc, x0, dinvb, Wt, Rt, bt, W3, b3, W4, b4, Wr, br):
    nb = NP // BN
    return pl.pallas_call(
        _final_body,
        grid=(nb,),
        in_specs=[
            pl.BlockSpec((KP, BN, W2H), lambda i: (0, i, 0)),
            pl.BlockSpec((BN, H), lambda i: (i, 0)),
            pl.BlockSpec((BN, H), lambda i: (i, 0)),
            pl.BlockSpec((K, H, H), lambda i: (0, 0, 0)),
            pl.BlockSpec((K, H, H), lambda i: (0, 0, 0)),
            pl.BlockSpec((K, 1, H), lambda i: (0, 0, 0)),
            pl.BlockSpec((H, H), lambda i: (0, 0)),
            pl.BlockSpec((1, H), lambda i: (0, 0)),
            pl.BlockSpec((H, H), lambda i: (0, 0)),
            pl.BlockSpec((1, H), lambda i: (0, 0)),
            pl.BlockSpec((H, D_OUT), lambda i: (0, 0)),
            pl.BlockSpec((1, D_OUT), lambda i: (0, 0)),
        ],
        out_specs=pl.BlockSpec((BN, D_OUT), lambda i: (i, 0)),
        out_shape=jax.ShapeDtypeStruct((NP, D_OUT), jnp.float32),
    )(acc, x0, dinvb, Wt, Rt, bt, W3, b3, W4, b4, Wr, br)


def kernel(x, edge_index, W1, b1, W2, b2, init_w, arma_w, root_w, arma_b,
           W3, b3, W4, b4, Wr, br):
    src = edge_index[0]
    dst = edge_index[1]

    pad16 = NC16 * C - EP16
    src16 = jnp.pad(src.reshape(TILES, EP16), ((0, 0), (0, pad16))
                    ).reshape(TILES, NBLK, BLK2, C)
    dst16 = jnp.pad(dst.reshape(TILES, EP16), ((0, 0), (0, pad16)),
                    constant_values=N).reshape(TILES, NBLK, BLK2, C)
    pad32 = NC32 * C - EP32
    src32 = jnp.pad(src.reshape(WORKERS, EP32), ((0, 0), (0, pad32))
                    ).reshape(WORKERS, NBLK32, BLK, C)
    dst32 = jnp.pad(dst.reshape(WORKERS, EP32), ((0, 0), (0, pad32)),
                    constant_values=N).reshape(WORKERS, NBLK32, BLK, C)

    xp = jnp.pad(x, ((0, NP - N), (0, 0)))
    z128 = jnp.zeros((ZR, W2H), jnp.float32)
    z16 = jnp.zeros((ZR, 16), jnp.float32)
    ones16 = jnp.ones((C, 16), jnp.float32)
    b1r = b1.reshape(1, H)
    b2r = b2.reshape(1, H)
    b3r = b3.reshape(1, H)
    b4r = b4.reshape(1, H)
    brr = br.reshape(1, D_OUT)

    deg2 = _sc_degree(dst32, ones16, z16)
    x0, curs, dinvb = _tc_pre(xp, W1, b1r, W2, b2r, deg2)
    acc0 = _sc_prop0(curs, src32, dst32, z128)
    cur = _tc_stage(acc0, x0, dinvb, init_w, root_w[0], arma_b[0], first=True)
    for t in range(1, T - 1):
        acc = _sc_prop(cur, src16, dst16, z128)
        cur = _tc_stage(acc, x0, dinvb, arma_w[t - 1], root_w[t], arma_b[t],
                        first=False)
    acc = _sc_prop(cur, src16, dst16, z128)
    y = _tc_final(acc, x0, dinvb, arma_w[T - 2], root_w[T - 1], arma_b[T - 1],
                  W3, b3r, W4, b4r, Wr, brr)
    return y[:N]
